# Initial kernel scaffold; baseline (speedup 1.0000x reference)
import jax
import jax.numpy as jnp
from jax.experimental import pallas as pl


def kernel(op_gid, cbo, enc, op_table):
    raise NotImplementedError("write your pallas kernel here")


# speedup vs baseline: 1.3571x; 1.3571x over previous
import functools

import jax
import jax.numpy as jnp
from jax import lax
from jax.experimental import pallas as pl
from jax.experimental.pallas import tpu as pltpu
from jax.experimental.pallas import tpu_sc as plsc

N = 100000
D_EMB = 32
D_CBO = 16
D_ENC = 80
D_OUT = D_EMB + D_CBO + D_ENC

BLK = 128
N_FULL = N // BLK
TAIL = N - N_FULL * BLK
TAIL_BASE = N_FULL * BLK

_info = plsc.get_sparse_core_info()
NC = _info.num_cores
NS = _info.num_subcores
NW = NC * NS
BLOCKS_PER_W = (N_FULL + NW - 1) // NW

_mesh = plsc.VectorSubcoreMesh(core_axis_name="c", subcore_axis_name="s")


@functools.partial(
    pl.kernel,
    mesh=_mesh,
    out_type=jax.ShapeDtypeStruct((N, D_OUT), jnp.float32),
    scratch_types=[
        pltpu.VMEM((BLK,), jnp.int32),
        pltpu.VMEM((BLK, D_OUT), jnp.float32),
        pltpu.VMEM((BLK, D_CBO), jnp.float32),
        pltpu.VMEM((BLK, D_ENC), jnp.float32),
        pltpu.SemaphoreType.DMA,
    ],
)
def _embed(gid, cbo, enc, table_pad, out, idx_v, row_v, cbo_v, enc_v, sem):
    wid = lax.axis_index("s") * NC + lax.axis_index("c")

    def do_block(base, rows):
        pltpu.sync_copy(gid.at[pl.ds(base, rows)], idx_v.at[pl.ds(0, rows)])
        pltpu.async_copy(table_pad.at[idx_v.at[pl.ds(0, rows)]],
                         row_v.at[pl.ds(0, rows)], sem).wait()
        pltpu.sync_copy(cbo.at[pl.ds(base, rows)], cbo_v.at[pl.ds(0, rows)])
        pltpu.sync_copy(enc.at[pl.ds(base, rows)], enc_v.at[pl.ds(0, rows)])

        def merge(r, _):
            row_v[r, pl.ds(D_EMB, D_CBO)] = cbo_v[r, :]
            for j in range(D_ENC // 16):
                row_v[r, pl.ds(D_EMB + D_CBO + j * 16, 16)] = \
                    enc_v[r, pl.ds(j * 16, 16)]
            return _

        lax.fori_loop(0, rows, merge, None)
        pltpu.sync_copy(row_v.at[pl.ds(0, rows)], out.at[pl.ds(base, rows)])

    def body(t, _):
        blk = wid + t * NW

        @pl.when(blk < N_FULL)
        def _():
            do_block(blk * BLK, BLK)

        return _

    lax.fori_loop(0, BLOCKS_PER_W, body, None)

    @pl.when(wid == NW - 1)
    def _tail():
        do_block(TAIL_BASE, TAIL)


def kernel(op_gid, cbo, enc, op_table):
    table_pad = jnp.pad(op_table, ((0, 0), (0, D_OUT - D_EMB)))
    return _embed(op_gid.astype(jnp.int32), cbo, enc, table_pad)


# speedup vs baseline: 1.6001x; 1.1790x over previous
import functools

import jax
import jax.numpy as jnp
from jax import lax
from jax.experimental import pallas as pl
from jax.experimental.pallas import tpu as pltpu
from jax.experimental.pallas import tpu_sc as plsc

N = 100000
D_EMB = 32
D_CBO = 16
D_ENC = 80
D_OUT = D_EMB + D_CBO + D_ENC

BLK = 128
SPAN = 3200
NBLK_MAIN = SPAN // BLK
NBLK_LAST = 6
TAIL = 32
TAIL_OFF = NBLK_LAST * BLK

_info = plsc.get_sparse_core_info()
NC = _info.num_cores
NS = _info.num_subcores
NW = NC * NS

_mesh = plsc.VectorSubcoreMesh(core_axis_name="c", subcore_axis_name="s")


@functools.partial(
    pl.kernel,
    mesh=_mesh,
    out_type=jax.ShapeDtypeStruct((N, D_OUT), jnp.float32),
    scratch_types=[
        pltpu.VMEM((SPAN,), jnp.int32),
        pltpu.VMEM((BLK, D_OUT), jnp.float32),
        pltpu.VMEM((BLK, D_OUT), jnp.float32),
        pltpu.VMEM((BLK, D_CBO), jnp.float32),
        pltpu.VMEM((BLK, D_CBO), jnp.float32),
        pltpu.VMEM((BLK, D_ENC), jnp.float32),
        pltpu.VMEM((BLK, D_ENC), jnp.float32),
        pltpu.SemaphoreType.DMA,
        pltpu.SemaphoreType.DMA,
        pltpu.SemaphoreType.DMA,
        pltpu.SemaphoreType.DMA,
        pltpu.SemaphoreType.DMA,
        pltpu.SemaphoreType.DMA,
        pltpu.SemaphoreType.DMA,
        pltpu.SemaphoreType.DMA,
    ],
)
def _embed(gid, cbo, enc, table_pad, out,
           idx_all, row0, row1, cbo0, cbo1, enc0, enc1,
           gs0, gs1, cs0, cs1, es0, es1, os0, os1):
    wid = lax.axis_index("s") * NC + lax.axis_index("c")
    base_w = wid * SPAN
    nblk = jnp.where(wid == NW - 1, NBLK_LAST, NBLK_MAIN)

    rows = (row0, row1)
    cbos = (cbo0, cbo1)
    encs = (enc0, enc1)
    gss = (gs0, gs1)
    css = (cs0, cs1)
    ess = (es0, es1)
    oss = (os0, os1)

    def issue_inputs(t, p):
        pltpu.async_copy(table_pad.at[idx_all.at[pl.ds(t * BLK, BLK)]],
                         rows[p], gss[p])
        pltpu.async_copy(cbo.at[pl.ds(base_w + t * BLK, BLK)], cbos[p], css[p])
        pltpu.async_copy(enc.at[pl.ds(base_w + t * BLK, BLK)], encs[p], ess[p])

    def wait_inputs(t, p):
        pltpu.make_async_copy(table_pad.at[idx_all.at[pl.ds(t * BLK, BLK)]],
                              rows[p], gss[p]).wait()
        pltpu.make_async_copy(cbo.at[pl.ds(base_w + t * BLK, BLK)],
                              cbos[p], css[p]).wait()
        pltpu.make_async_copy(enc.at[pl.ds(base_w + t * BLK, BLK)],
                              encs[p], ess[p]).wait()

    def merge(p, nrows4):
        row_v, cbo_v, enc_v = rows[p], cbos[p], encs[p]

        def mbody(r4, _):
            for dr in range(4):
                r = r4 * 4 + dr
                row_v[r, pl.ds(D_EMB, D_CBO)] = cbo_v[r, :]
                for j in range(D_ENC // 16):
                    row_v[r, pl.ds(D_EMB + D_CBO + j * 16, 16)] = \
                        enc_v[r, pl.ds(j * 16, 16)]
            return _

        lax.fori_loop(0, nrows4, mbody, None)

    def issue_out(t, p):
        pltpu.async_copy(rows[p], out.at[pl.ds(base_w + t * BLK, BLK)], oss[p])

    def wait_out(t, p):
        pltpu.make_async_copy(rows[p], out.at[pl.ds(base_w + t * BLK, BLK)],
                              oss[p]).wait()

    @pl.when(wid < NW - 1)
    def _():
        pltpu.sync_copy(gid.at[pl.ds(base_w, SPAN)], idx_all)

    @pl.when(wid == NW - 1)
    def _():
        pltpu.sync_copy(gid.at[pl.ds(base_w, TAIL_OFF + TAIL)],
                        idx_all.at[pl.ds(0, TAIL_OFF + TAIL)])

    issue_inputs(0, 0)
    issue_inputs(1, 1)

    def pair(u, _):
        for h in (0, 1):
            t = u * 2 + h

            @pl.when(t < nblk)
            def _():
                wait_inputs(t, h)
                merge(h, BLK // 4)
                issue_out(t, h)

        for h in (0, 1):
            t_next = u * 2 + 2 + h

            @pl.when(t_next < nblk)
            def _():
                wait_out(t_next - 2, h)
                issue_inputs(t_next, h)

        return _

    lax.fori_loop(0, (NBLK_MAIN + 1) // 2, pair, None)

    @pl.when(wid < NW - 1)
    def _():
        wait_out(NBLK_MAIN - 2, (NBLK_MAIN - 2) % 2)
        wait_out(NBLK_MAIN - 1, (NBLK_MAIN - 1) % 2)

    @pl.when(wid == NW - 1)
    def _():
        wait_out(NBLK_LAST - 2, (NBLK_LAST - 2) % 2)
        wait_out(NBLK_LAST - 1, (NBLK_LAST - 1) % 2)

        pltpu.async_copy(table_pad.at[idx_all.at[pl.ds(TAIL_OFF, TAIL)]],
                         row0.at[pl.ds(0, TAIL)], gs0).wait()
        pltpu.sync_copy(cbo.at[pl.ds(base_w + TAIL_OFF, TAIL)],
                        cbo0.at[pl.ds(0, TAIL)])
        pltpu.sync_copy(enc.at[pl.ds(base_w + TAIL_OFF, TAIL)],
                        enc0.at[pl.ds(0, TAIL)])
        merge(0, TAIL // 4)
        pltpu.sync_copy(row0.at[pl.ds(0, TAIL)],
                        out.at[pl.ds(base_w + TAIL_OFF, TAIL)])


def kernel(op_gid, cbo, enc, op_table):
    table_pad = jnp.pad(op_table, ((0, 0), (0, D_OUT - D_EMB)))
    return _embed(op_gid.astype(jnp.int32), cbo, enc, table_pad)
